# Initial kernel scaffold; baseline (speedup 1.0000x reference)
import jax
import jax.numpy as jnp
from jax.experimental import pallas as pl


def kernel(h, edge_index, b2_edges, b2_cells, We1, be1, We2, be2, Wc1, bc1, Wc2, bc2, Wn1, bn1, Wn2, bn2, Wg, bg):
    raise NotImplementedError("write your pallas kernel here")


# speedup vs baseline: 1.0200x; 1.0200x over previous
import functools

import jax
import jax.numpy as jnp
from jax.experimental import pallas as pl
from jax.experimental.pallas import tpu as pltpu

H = 256


def _pick_blk(m):
    for b in (1024, 800, 512, 400, 320, 256, 160, 128, 80, 64, 8):
        if m % b == 0:
            return b
    raise ValueError(m)


def _mlp_body(x_ref, d_ref, w1_ref, b1_ref, w2_ref, b2_ref, o_ref):
    x = x_ref[...] * d_ref[...]
    x = jnp.maximum(x, 0.0)
    h1 = jnp.maximum(
        jax.lax.dot(x, w1_ref[...], preferred_element_type=jnp.float32)
        + b1_ref[...],
        0.0,
    )
    o_ref[...] = (
        jax.lax.dot(h1, w2_ref[...], preferred_element_type=jnp.float32)
        + b2_ref[...]
    )


def _fused_mlp(x, inv_deg, w1, b1, w2, b2):
    m = x.shape[0]
    BLK = _pick_blk(m)
    grid = m // BLK
    return pl.pallas_call(
        _mlp_body,
        grid=(grid,),
        in_specs=[
            pl.BlockSpec((BLK, H), lambda i: (i, 0)),
            pl.BlockSpec((BLK, 1), lambda i: (i, 0)),
            pl.BlockSpec((H, H), lambda i: (0, 0)),
            pl.BlockSpec((1, H), lambda i: (0, 0)),
            pl.BlockSpec((H, H), lambda i: (0, 0)),
            pl.BlockSpec((1, H), lambda i: (0, 0)),
        ],
        out_specs=pl.BlockSpec((BLK, H), lambda i: (i, 0)),
        out_shape=jax.ShapeDtypeStruct((m, H), jnp.float32),
    )(x, inv_deg, w1, b1, w2, b2)


def _tail_body(x_ref, d_ref, hb_ref, w1_ref, b1_ref, w2_ref, b2_ref,
               wg1_ref, wg2_ref, bg_ref, o_ref):
    x = x_ref[...] * d_ref[...]
    x = jnp.maximum(x, 0.0)
    h1 = jnp.maximum(
        jax.lax.dot(x, w1_ref[...], preferred_element_type=jnp.float32)
        + b1_ref[...],
        0.0,
    )
    hm = (
        jax.lax.dot(h1, w2_ref[...], preferred_element_type=jnp.float32)
        + b2_ref[...]
    )
    hb = hb_ref[...]
    zlin = (
        jax.lax.dot(hb, wg1_ref[...], preferred_element_type=jnp.float32)
        + jax.lax.dot(hm, wg2_ref[...], preferred_element_type=jnp.float32)
        + bg_ref[...]
    )
    z = jax.nn.sigmoid(zlin)
    o_ref[...] = hb + z * hm


def _fused_tail(x, inv_deg, hb, w1, b1, w2, b2, wg1, wg2, bg):
    m = x.shape[0]
    BLK = _pick_blk(m)
    grid = m // BLK
    full = lambda i: (0, 0)
    return pl.pallas_call(
        _tail_body,
        grid=(grid,),
        in_specs=[
            pl.BlockSpec((BLK, H), lambda i: (i, 0)),
            pl.BlockSpec((BLK, 1), lambda i: (i, 0)),
            pl.BlockSpec((BLK, H), lambda i: (i, 0)),
            pl.BlockSpec((H, H), full),
            pl.BlockSpec((1, H), full),
            pl.BlockSpec((H, H), full),
            pl.BlockSpec((1, H), full),
            pl.BlockSpec((H, H), full),
            pl.BlockSpec((H, H), full),
            pl.BlockSpec((1, H), full),
        ],
        out_specs=pl.BlockSpec((BLK, H), lambda i: (i, 0)),
        out_shape=jax.ShapeDtypeStruct((m, H), jnp.float32),
    )(x, inv_deg, hb, w1, b1, w2, b2, wg1, wg2, bg)


def kernel(h, edge_index, b2_edges, b2_cells, We1, be1, We2, be2,
           Wc1, bc1, Wc2, bc2, Wn1, bn1, Wn2, bn2, Wg, bg):
    B, N, _ = h.shape
    E = edge_index.shape[1]
    C = 40000
    src = edge_index[0]
    dst = edge_index[1]

    deg_nodes = jnp.maximum(
        jnp.zeros((N,), jnp.float32).at[src].add(1.0).at[dst].add(1.0), 1.0)
    deg_cells = jnp.maximum(
        jnp.zeros((C,), jnp.float32).at[b2_cells].add(1.0), 1.0)
    deg_edges_c = jnp.maximum(
        jnp.zeros((E,), jnp.float32).at[b2_edges].add(1.0), 1.0)

    inv_dn = (1.0 / deg_nodes)[:, None]
    inv_dc = (1.0 / deg_cells)[:, None]
    inv_dec = (1.0 / deg_edges_c)[:, None]

    b1r = lambda v: v[None, :]

    x_e = h[:, dst, :] - h[:, src, :]
    half = jnp.full((B * E, 1), 0.5, jnp.float32)
    e_mlp = _fused_mlp(x_e.reshape(B * E, H), half, We1, b1r(be1), We2,
                       b1r(be2)).reshape(B, E, H)

    c_raw = jnp.zeros((B, C, H), jnp.float32).at[:, b2_cells, :].add(
        e_mlp[:, b2_edges, :])
    c_mlp = _fused_mlp(c_raw.reshape(B * C, H),
                       jnp.tile(inv_dc, (B, 1)), Wc1, b1r(bc1), Wc2,
                       b1r(bc2)).reshape(B, C, H)

    e2 = jnp.zeros((B, E, H), jnp.float32).at[:, b2_edges, :].add(
        c_mlp[:, b2_cells, :])
    m = e_mlp + e2 * inv_dec[None]
    hm_raw = (jnp.zeros((B, N, H), jnp.float32)
              .at[:, dst, :].add(m).at[:, src, :].add(-m))

    out = _fused_tail(hm_raw.reshape(B * N, H), jnp.tile(inv_dn, (B, 1)),
                      h.reshape(B * N, H), Wn1, b1r(bn1), Wn2, b1r(bn2),
                      Wg[:H], Wg[H:], b1r(bg))
    return out.reshape(B, N, H)


# speedup vs baseline: 1.0911x; 1.0697x over previous
import functools

import jax
import jax.numpy as jnp
from jax import lax
from jax.experimental import pallas as pl
from jax.experimental.pallas import tpu as pltpu
from jax.experimental.pallas import tpu_sc as plsc

H = 256
N = 10000
E = 160000
C = 40000
B = 4

NW = 32
EPW = 5120
E_PAD = EPW * NW
R_ROWS = 320
N_PAD = R_ROWS * NW
C_PAD = R_ROWS * NW * 4
SENT = 1 << 30

SC_CHUNK = 512
G = 64
CBUF = SC_CHUNK + G + 16


def _pick_blk(m):
    for b in (1024, 800, 512, 400, 320, 256, 160, 128, 80, 64, 8):
        if m % b == 0:
            return b
    raise ValueError(m)


def _mlp_body(x_ref, d_ref, w1_ref, b1_ref, w2_ref, b2_ref, o_ref):
    x = jnp.maximum(x_ref[...] * d_ref[...], 0.0)
    h1 = jnp.maximum(
        jax.lax.dot(x, w1_ref[...], preferred_element_type=jnp.float32)
        + b1_ref[...], 0.0)
    o_ref[...] = (
        jax.lax.dot(h1, w2_ref[...], preferred_element_type=jnp.float32)
        + b2_ref[...])


def _fused_mlp(x, inv_deg, w1, b1, w2, b2):
    m = x.shape[0]
    blk = _pick_blk(m)
    return pl.pallas_call(
        _mlp_body,
        grid=(m // blk,),
        in_specs=[
            pl.BlockSpec((blk, H), lambda i: (i, 0)),
            pl.BlockSpec((blk, 1), lambda i: (i, 0)),
            pl.BlockSpec((H, H), lambda i: (0, 0)),
            pl.BlockSpec((1, H), lambda i: (0, 0)),
            pl.BlockSpec((H, H), lambda i: (0, 0)),
            pl.BlockSpec((1, H), lambda i: (0, 0)),
        ],
        out_specs=pl.BlockSpec((blk, H), lambda i: (i, 0)),
        out_shape=jax.ShapeDtypeStruct((m, H), jnp.float32),
    )(x, inv_deg, w1, b1, w2, b2)


def _tail_body(x_ref, d_ref, hb_ref, w1_ref, b1_ref, w2_ref, b2_ref,
               wg1_ref, wg2_ref, bg_ref, o_ref):
    x = jnp.maximum(x_ref[...] * d_ref[...], 0.0)
    h1 = jnp.maximum(
        jax.lax.dot(x, w1_ref[...], preferred_element_type=jnp.float32)
        + b1_ref[...], 0.0)
    hm = (jax.lax.dot(h1, w2_ref[...], preferred_element_type=jnp.float32)
          + b2_ref[...])
    hb = hb_ref[...]
    z = jax.nn.sigmoid(
        jax.lax.dot(hb, wg1_ref[...], preferred_element_type=jnp.float32)
        + jax.lax.dot(hm, wg2_ref[...], preferred_element_type=jnp.float32)
        + bg_ref[...])
    o_ref[...] = hb + z * hm


def _fused_tail(x, inv_deg, hb, w1, b1, w2, b2, wg1, wg2, bg):
    m = x.shape[0]
    blk = _pick_blk(m)
    full = lambda i: (0, 0)
    return pl.pallas_call(
        _tail_body,
        grid=(m // blk,),
        in_specs=[
            pl.BlockSpec((blk, H), lambda i: (i, 0)),
            pl.BlockSpec((blk, 1), lambda i: (i, 0)),
            pl.BlockSpec((blk, H), lambda i: (i, 0)),
            pl.BlockSpec((H, H), full),
            pl.BlockSpec((1, H), full),
            pl.BlockSpec((H, H), full),
            pl.BlockSpec((1, H), full),
            pl.BlockSpec((H, H), full),
            pl.BlockSpec((H, H), full),
            pl.BlockSpec((1, H), full),
        ],
        out_specs=pl.BlockSpec((blk, H), lambda i: (i, 0)),
        out_shape=jax.ShapeDtypeStruct((m, H), jnp.float32),
    )(x, inv_deg, hb, w1, b1, w2, b2, wg1, wg2, bg)


def _sc_mesh():
    return plsc.VectorSubcoreMesh(core_axis_name="c", subcore_axis_name="s")


_SC_PARAMS = dict(
    compiler_params=pltpu.CompilerParams(needs_layout_passes=False))


def _wid():
    return lax.axis_index("s") * 2 + lax.axis_index("c")


def _edge_gather(h_flat, didx, sidx):
    KC = 128

    @functools.partial(
        pl.kernel,
        out_type=jax.ShapeDtypeStruct((B * E_PAD, H), jnp.float32),
        mesh=_sc_mesh(),
        scratch_types=[
            pltpu.VMEM((KC,), jnp.int32),
            pltpu.VMEM((KC,), jnp.int32),
            pltpu.VMEM((KC,), jnp.int32),
            pltpu.VMEM((KC,), jnp.int32),
            pltpu.VMEM((KC, H), jnp.float32),
            pltpu.VMEM((KC, H), jnp.float32),
            pltpu.SemaphoreType.DMA,
            pltpu.SemaphoreType.DMA,
        ],
        **_SC_PARAMS,
    )
    def k(h_hbm, d_hbm, s_hbm, out_hbm, di_v, si_v, ib1_v, ib2_v, rd_v,
          rs_v, sem1, sem2):
        w = _wid()
        ebase = w * EPW

        def chunk(ch, _):
            off = pl.multiple_of(ebase + ch * KC, 8)
            pltpu.sync_copy(d_hbm.at[pl.ds(off, KC)], di_v)
            pltpu.sync_copy(s_hbm.at[pl.ds(off, KC)], si_v)

            def per_b(b, _):
                for kk in range(KC // 16):
                    ib1_v[pl.ds(kk * 16, 16)] = (
                        di_v[pl.ds(kk * 16, 16)] + b * N)
                    ib2_v[pl.ds(kk * 16, 16)] = (
                        si_v[pl.ds(kk * 16, 16)] + b * N)
                cp1 = pltpu.async_copy(h_hbm.at[ib1_v], rd_v, sem1)
                cp2 = pltpu.async_copy(h_hbm.at[ib2_v], rs_v, sem2)
                cp1.wait()
                cp2.wait()

                def row(rr, _):
                    for j in range(H // 16):
                        d = rd_v[rr, pl.ds(j * 16, 16)]
                        s = rs_v[rr, pl.ds(j * 16, 16)]
                        rd_v[rr, pl.ds(j * 16, 16)] = jnp.maximum(
                            (d - s) * 0.5, 0.0)
                    return 0

                lax.fori_loop(0, KC, row, 0)
                orow = b * E_PAD + ebase + ch * KC
                pltpu.sync_copy(rd_v, out_hbm.at[pl.ds(orow, KC)])
                return 0

            lax.fori_loop(0, B, per_b, 0)
            return 0

        lax.fori_loop(0, EPW // KC, chunk, 0)

    return k(h_flat, didx, sidx)


def _accumulate(table, mask_ids, rid_ids, w_arr, sign, n_pass, tbl_rows,
                out_rows, use_pos_rid, phase_tag):
    have_w = w_arr is not None
    have_rid = rid_ids is not None
    scr = [
        pltpu.VMEM((SC_CHUNK,), jnp.int32),
        pltpu.VMEM((SC_CHUNK,), jnp.int32),
        pltpu.VMEM((SC_CHUNK,), jnp.float32),
        pltpu.VMEM((CBUF,), jnp.int32),
        pltpu.VMEM((CBUF,), jnp.int32),
        pltpu.VMEM((CBUF,), jnp.float32),
        pltpu.VMEM((G,), jnp.int32),
        pltpu.VMEM((G, H), jnp.float32),
        pltpu.VMEM(((R_ROWS + 1) * H,), jnp.float32),
        pltpu.SemaphoreType.DMA,
    ]

    @functools.partial(
        pl.kernel,
        out_type=jax.ShapeDtypeStruct((B * out_rows * H,), jnp.float32),
        mesh=_sc_mesh(),
        scratch_types=scr,
        name=f"sc_accum_{phase_tag}",
        **_SC_PARAMS,
    )
    def k(tab_hbm, mid_hbm, rid_hbm, w_hbm, out_hbm, stm_v, str_v, stw_v,
          cd_v, cr_v, cw_v, gi_v, rows_v, acc_v, sem):
        w = _wid()
        iota = lax.iota(jnp.int32, 16)

        def per_pass(p, _):
            lo = (p * NW + w) * R_ROWS

            def per_b(b, _):
                def z(i, _):
                    acc_v[pl.ds(pl.multiple_of(i * 16, 8), 16)] = (
                        jnp.zeros((16,), jnp.float32))
                    return 0
                lax.fori_loop(0, (R_ROWS + 1) * H // 16, z, 0)

                def chunk(sc, _):
                    off = pl.multiple_of(sc * SC_CHUNK, 8)
                    pltpu.sync_copy(mid_hbm.at[pl.ds(off, SC_CHUNK)], stm_v)
                    if have_rid:
                        pltpu.sync_copy(
                            rid_hbm.at[pl.ds(off, SC_CHUNK)], str_v)
                    if have_w:
                        pltpu.sync_copy(
                            w_hbm.at[pl.ds(off, SC_CHUNK)], stw_v)

                    def comp(g, cnt):
                        o = pl.multiple_of(g * 16, 8)
                        v = stm_v[pl.ds(o, 16)]
                        d = v - lo
                        m = (d >= 0) & (d < R_ROWS)
                        mi = jnp.where(m, 1, 0)
                        pos = cnt + plsc.cumsum(mi) - 1
                        plsc.store_scatter(cd_v, [pos], d, mask=m)
                        if have_rid:
                            r = str_v[pl.ds(o, 16)] + b * tbl_rows
                        else:
                            r = (off + o + iota) + b * tbl_rows
                        plsc.store_scatter(cr_v, [pos], r, mask=m)
                        if have_w:
                            plsc.store_scatter(
                                cw_v, [pos], stw_v[pl.ds(o, 16)], mask=m)
                        return cnt + jnp.sum(mi)

                    cnt = lax.fori_loop(0, SC_CHUNK // 16, comp, 0)

                    for kk in range(G // 16):
                        pp = cnt + kk * 16 + iota
                        plsc.store_scatter(
                            cd_v, [pp], jnp.full((16,), R_ROWS, jnp.int32))
                        plsc.store_scatter(
                            cr_v, [pp], (w * 16 + iota) & 63)
                        if have_w:
                            plsc.store_scatter(
                                cw_v, [pp], jnp.zeros((16,), jnp.float32))

                    ng = (cnt + (G - 1)) >> 6

                    def grp(g, _):
                        gb = g * G
                        for kk in range(G // 16):
                            gi_v[pl.ds(kk * 16, 16)] = plsc.load_gather(
                                cr_v, [gb + kk * 16 + iota])
                        pltpu.async_copy(
                            tab_hbm.at[gi_v], rows_v, sem).wait()

                        def row(rr, _):
                            d16 = plsc.load_gather(
                                cd_v, [jnp.full((16,), gb + rr, jnp.int32)])
                            dbase = d16 * H + iota
                            if have_w:
                                w16 = plsc.load_gather(
                                    cw_v,
                                    [jnp.full((16,), gb + rr, jnp.int32)])
                                scale = w16 * sign
                            else:
                                scale = None
                            for j in range(H // 16):
                                seg = rows_v[rr, pl.ds(j * 16, 16)]
                                if scale is not None:
                                    seg = seg * scale
                                elif sign != 1.0:
                                    seg = seg * sign
                                plsc.addupdate_scatter(
                                    acc_v, [dbase + j * 16], seg)
                            return 0

                        lax.fori_loop(0, G, row, 0)
                        return 0

                    lax.fori_loop(0, ng, grp, 0)
                    return 0

                lax.fori_loop(0, E_PAD // SC_CHUNK, chunk, 0)
                obase = pl.multiple_of(
                    (b * out_rows + (p * NW + w) * R_ROWS) * H, 8)
                pltpu.sync_copy(
                    acc_v.at[pl.ds(0, R_ROWS * H)],
                    out_hbm.at[pl.ds(obase, R_ROWS * H)])
                return 0

            lax.fori_loop(0, B, per_b, 0)
            return 0

        lax.fori_loop(0, n_pass, per_pass, 0)

    rid_in = rid_ids if have_rid else jnp.zeros((8,), jnp.int32)
    w_in = w_arr if have_w else jnp.zeros((8,), jnp.float32)
    return k(table, mask_ids, rid_in, w_in)


def kernel(h, edge_index, b2_edges, b2_cells, We1, be1, We2, be2,
           Wc1, bc1, Wc2, bc2, Wn1, bn1, Wn2, bn2, Wg, bg):
    src = edge_index[0]
    dst = edge_index[1]

    deg_nodes = jnp.maximum(
        jnp.zeros((N,), jnp.float32).at[src].add(1.0).at[dst].add(1.0), 1.0)
    deg_cells = jnp.maximum(
        jnp.zeros((C,), jnp.float32).at[b2_cells].add(1.0), 1.0)
    deg_edges_c = jnp.maximum(
        jnp.zeros((E,), jnp.float32).at[b2_edges].add(1.0), 1.0)
    inv_dn = (1.0 / deg_nodes)[:, None]
    inv_dc = 1.0 / deg_cells
    wp = (1.0 / deg_edges_c)[b2_edges]
    pd = dst[b2_edges]
    ps = src[b2_edges]

    pad_iota = jnp.arange(E_PAD - E, dtype=jnp.int32)
    padN = lambda a: jnp.concatenate([a, pad_iota % N])
    padS = lambda a: jnp.concatenate(
        [a, jnp.full((E_PAD - E,), SENT, jnp.int32)])
    pad0i = lambda a: jnp.concatenate(
        [a, jnp.zeros((E_PAD - E,), jnp.int32)])
    pad0f = lambda a: jnp.concatenate(
        [a, jnp.zeros((E_PAD - E,), jnp.float32)])

    dst_p, src_p = padN(dst), padN(src)
    dst_m, src_m = padS(dst), padS(src)
    cells_m = padS(b2_cells)
    edges_r = pad0i(b2_edges)
    cells_r = pad0i(b2_cells)
    pd_m, ps_m = padS(pd), padS(ps)
    wp_p = pad0f(wp)

    b1r = lambda v: v[None, :]

    x_e = _edge_gather(h.reshape(B * N, H), dst_p, src_p)
    ones = jnp.ones((B * E_PAD, 1), jnp.float32)
    e_mlp = _fused_mlp(x_e.reshape(B * E_PAD, H), ones, We1, b1r(be1),
                       We2, b1r(be2))

    c_raw = _accumulate(e_mlp, cells_m, edges_r, None, 1.0, 4, E_PAD,
                        C_PAD, False, "cells")
    inv_dc_p = jnp.concatenate(
        [inv_dc, jnp.ones((C_PAD - C,), jnp.float32)])
    c_mlp = _fused_mlp(c_raw.reshape(B * C_PAD, H),
                       jnp.tile(inv_dc_p[:, None], (B, 1)),
                       Wc1, b1r(bc1), Wc2, b1r(bc2))

    hm1 = _accumulate(e_mlp, dst_m, None, None, 1.0, 1, E_PAD, N_PAD,
                      True, "n_e_dst")
    hm2 = _accumulate(e_mlp, src_m, None, None, -1.0, 1, E_PAD, N_PAD,
                      True, "n_e_src")
    hm3 = _accumulate(c_mlp, pd_m, cells_r, wp_p, 1.0, 1, C_PAD, N_PAD,
                      False, "n_c_dst")
    hm4 = _accumulate(c_mlp, ps_m, cells_r, wp_p, -1.0, 1, C_PAD, N_PAD,
                      False, "n_c_src")
    hm_raw = (hm1 + hm2 + hm3 + hm4).reshape(B, N_PAD, H)[:, :N, :]

    out = _fused_tail(hm_raw.reshape(B * N, H), jnp.tile(inv_dn, (B, 1)),
                      h.reshape(B * N, H), Wn1, b1r(bn1), Wn2, b1r(bn2),
                      Wg[:H], Wg[H:], b1r(bg))
    return out.reshape(B, N, H)


# speedup vs baseline: 3.9639x; 3.6328x over previous
import functools

import jax
import jax.numpy as jnp
from jax import lax
from jax.experimental import pallas as pl
from jax.experimental.pallas import tpu as pltpu
from jax.experimental.pallas import tpu_sc as plsc

H = 256
N = 10000
E = 160000
C = 40000
B = 4

NW = 32
EPW = 5120
E_PAD = EPW * NW
R_ROWS = 320
N_PAD = R_ROWS * NW
C_PAD = R_ROWS * NW * 4
SENT = 1 << 30

SC_CHUNK = 2048
G = 64
CBUF = SC_CHUNK + 2 * G


def _pick_blk(m):
    for b in (1024, 800, 512, 400, 320, 256, 160, 128, 80, 64, 8):
        if m % b == 0:
            return b
    raise ValueError(m)


def _mlp_body(x_ref, d_ref, w1_ref, b1_ref, w2_ref, b2_ref, o_ref):
    x = jnp.maximum(x_ref[...] * d_ref[...], 0.0)
    h1 = jnp.maximum(
        jax.lax.dot(x, w1_ref[...], preferred_element_type=jnp.float32)
        + b1_ref[...], 0.0)
    o_ref[...] = (
        jax.lax.dot(h1, w2_ref[...], preferred_element_type=jnp.float32)
        + b2_ref[...])


def _fused_mlp(x, inv_deg, w1, b1, w2, b2):
    m = x.shape[0]
    blk = _pick_blk(m)
    return pl.pallas_call(
        _mlp_body,
        grid=(m // blk,),
        in_specs=[
            pl.BlockSpec((blk, H), lambda i: (i, 0)),
            pl.BlockSpec((blk, 1), lambda i: (i, 0)),
            pl.BlockSpec((H, H), lambda i: (0, 0)),
            pl.BlockSpec((1, H), lambda i: (0, 0)),
            pl.BlockSpec((H, H), lambda i: (0, 0)),
            pl.BlockSpec((1, H), lambda i: (0, 0)),
        ],
        out_specs=pl.BlockSpec((blk, H), lambda i: (i, 0)),
        out_shape=jax.ShapeDtypeStruct((m, H), jnp.float32),
    )(x, inv_deg, w1, b1, w2, b2)


def _tail_body(x_ref, d_ref, hb_ref, w1_ref, b1_ref, w2_ref, b2_ref,
               wg1_ref, wg2_ref, bg_ref, o_ref):
    x = jnp.maximum(x_ref[...] * d_ref[...], 0.0)
    h1 = jnp.maximum(
        jax.lax.dot(x, w1_ref[...], preferred_element_type=jnp.float32)
        + b1_ref[...], 0.0)
    hm = (jax.lax.dot(h1, w2_ref[...], preferred_element_type=jnp.float32)
          + b2_ref[...])
    hb = hb_ref[...]
    z = jax.nn.sigmoid(
        jax.lax.dot(hb, wg1_ref[...], preferred_element_type=jnp.float32)
        + jax.lax.dot(hm, wg2_ref[...], preferred_element_type=jnp.float32)
        + bg_ref[...])
    o_ref[...] = hb + z * hm


def _fused_tail(x, inv_deg, hb, w1, b1, w2, b2, wg1, wg2, bg):
    m = x.shape[0]
    blk = _pick_blk(m)
    full = lambda i: (0, 0)
    return pl.pallas_call(
        _tail_body,
        grid=(m // blk,),
        in_specs=[
            pl.BlockSpec((blk, H), lambda i: (i, 0)),
            pl.BlockSpec((blk, 1), lambda i: (i, 0)),
            pl.BlockSpec((blk, H), lambda i: (i, 0)),
            pl.BlockSpec((H, H), full),
            pl.BlockSpec((1, H), full),
            pl.BlockSpec((H, H), full),
            pl.BlockSpec((1, H), full),
            pl.BlockSpec((H, H), full),
            pl.BlockSpec((H, H), full),
            pl.BlockSpec((1, H), full),
        ],
        out_specs=pl.BlockSpec((blk, H), lambda i: (i, 0)),
        out_shape=jax.ShapeDtypeStruct((m, H), jnp.float32),
    )(x, inv_deg, hb, w1, b1, w2, b2, wg1, wg2, bg)


def _sc_mesh():
    return plsc.VectorSubcoreMesh(core_axis_name="c", subcore_axis_name="s")


_SC_PARAMS = dict(
    compiler_params=pltpu.CompilerParams(needs_layout_passes=False))


def _wid():
    return lax.axis_index("s") * 2 + lax.axis_index("c")


def _edge_gather(h_flat, didx, sidx):
    KC = 128

    @functools.partial(
        pl.kernel,
        out_type=jax.ShapeDtypeStruct((B * E_PAD, H), jnp.float32),
        mesh=_sc_mesh(),
        scratch_types=[
            pltpu.VMEM((KC,), jnp.int32),
            pltpu.VMEM((KC,), jnp.int32),
            pltpu.VMEM((KC,), jnp.int32),
            pltpu.VMEM((KC,), jnp.int32),
            pltpu.VMEM((KC, H), jnp.float32),
            pltpu.VMEM((KC, H), jnp.float32),
            pltpu.SemaphoreType.DMA,
            pltpu.SemaphoreType.DMA,
        ],
        **_SC_PARAMS,
    )
    def k(h_hbm, d_hbm, s_hbm, out_hbm, di_v, si_v, ib1_v, ib2_v, rd_v,
          rs_v, sem1, sem2):
        w = _wid()
        ebase = w * EPW

        def chunk(ch, _):
            off = pl.multiple_of(ebase + ch * KC, 8)
            pltpu.sync_copy(d_hbm.at[pl.ds(off, KC)], di_v)
            pltpu.sync_copy(s_hbm.at[pl.ds(off, KC)], si_v)

            def per_b(b, _):
                for kk in range(KC // 16):
                    ib1_v[pl.ds(kk * 16, 16)] = (
                        di_v[pl.ds(kk * 16, 16)] + b * N)
                    ib2_v[pl.ds(kk * 16, 16)] = (
                        si_v[pl.ds(kk * 16, 16)] + b * N)
                cp1 = pltpu.async_copy(h_hbm.at[ib1_v], rd_v, sem1)
                cp2 = pltpu.async_copy(h_hbm.at[ib2_v], rs_v, sem2)
                cp1.wait()
                cp2.wait()

                def row(rr, _):
                    for j in range(H // 16):
                        d = rd_v[rr, pl.ds(j * 16, 16)]
                        s = rs_v[rr, pl.ds(j * 16, 16)]
                        rd_v[rr, pl.ds(j * 16, 16)] = jnp.maximum(
                            (d - s) * 0.5, 0.0)
                    return 0

                lax.fori_loop(0, KC, row, 0)
                orow = b * E_PAD + ebase + ch * KC
                pltpu.sync_copy(rd_v, out_hbm.at[pl.ds(orow, KC)])
                return 0

            lax.fori_loop(0, B, per_b, 0)
            return 0

        lax.fori_loop(0, EPW // KC, chunk, 0)

    return k(h_flat, didx, sidx)


def _accumulate(tabs, phases, n_pass, out_rows, tag):
    n_tab = len(tabs)
    scr = [
        pltpu.VMEM((SC_CHUNK,), jnp.int32),
        pltpu.VMEM((SC_CHUNK,), jnp.int32),
        pltpu.VMEM((SC_CHUNK,), jnp.float32),
        pltpu.VMEM((CBUF,), jnp.int32),
        pltpu.VMEM((CBUF,), jnp.int32),
        pltpu.VMEM((CBUF,), jnp.float32),
        pltpu.VMEM((16,), jnp.int32),
        pltpu.VMEM((G,), jnp.int32),
        pltpu.VMEM((G, H), jnp.float32),
        pltpu.VMEM(((R_ROWS + 1) * H,), jnp.float32),
        pltpu.SemaphoreType.DMA,
        pltpu.SemaphoreType.DMA,
        pltpu.SemaphoreType.DMA,
        pltpu.SemaphoreType.DMA,
    ]

    @functools.partial(
        pl.kernel,
        out_type=jax.ShapeDtypeStruct((B * out_rows * H,), jnp.float32),
        mesh=_sc_mesh(),
        scratch_types=scr,
        name=f"sc_accum_{tag}",
        **_SC_PARAMS,
    )
    def k(*refs):
        tab_hbm = refs[:n_tab]
        ph_hbm = refs[n_tab:n_tab + 3 * len(phases)]
        out_hbm = refs[n_tab + 3 * len(phases)]
        (stm_v, str_v, stw_v, cd_v, cr_v, cw_v, cs_v, gi_v, rows_v,
         acc_v, sg, sem, semr, semw) = refs[n_tab + 3 * len(phases) + 1:]
        w = _wid()
        iota = lax.iota(jnp.int32, 16)

        def process(tab, ng, scale_kind, sign):
            def grp(g, _):
                gb = g * G
                for kk in range(G // 16):
                    gi_v[pl.ds(kk * 16, 16)] = plsc.load_gather(
                        cr_v, [gb + kk * 16 + iota])
                pltpu.async_copy(tab.at[gi_v], rows_v, sg).wait()

                def row(rr, _):
                    d16 = plsc.load_gather(
                        cd_v, [jnp.full((16,), gb + rr, jnp.int32)])
                    dbase = d16 * H + iota
                    if scale_kind:
                        w16 = plsc.load_gather(
                            cw_v, [jnp.full((16,), gb + rr, jnp.int32)])
                        scale = w16 * sign
                    else:
                        scale = None
                    for j in range(H // 16):
                        seg = rows_v[rr, pl.ds(j * 16, 16)]
                        if scale is not None:
                            seg = seg * scale
                        elif sign != 1.0:
                            seg = seg * sign
                        plsc.addupdate_scatter(acc_v, [dbase + j * 16], seg)
                    return 0

                lax.fori_loop(0, G, row, 0)
                return 0

            lax.fori_loop(0, ng, grp, 0)

        def carry_to_front(ng, cnt):
            base = ng * G
            for kk in range(G // 16):
                sp = base + kk * 16 + iota
                dp = kk * 16 + iota
                plsc.store_scatter(cd_v, [dp], plsc.load_gather(cd_v, [sp]))
                plsc.store_scatter(cr_v, [dp], plsc.load_gather(cr_v, [sp]))
                plsc.store_scatter(cw_v, [dp], plsc.load_gather(cw_v, [sp]))
            return cnt - base

        def per_pass(p, _):
            lo = (p * NW + w) * R_ROWS

            def per_b(b, _):
                def z(i, _):
                    acc_v[pl.ds(pl.multiple_of(i * 16, 8), 16)] = (
                        jnp.zeros((16,), jnp.float32))
                    return 0
                lax.fori_loop(0, (R_ROWS + 1) * H // 16, z, 0)

                for pi, (_, _, _, sign, ti, tbl_rows) in enumerate(phases):
                    mid_hbm = ph_hbm[3 * pi]
                    rid_hbm = ph_hbm[3 * pi + 1]
                    w_hbm = ph_hbm[3 * pi + 2]
                    have_rid = rid_hbm is not None and phases[pi][1] is not None
                    have_w = phases[pi][2] is not None
                    tab = tab_hbm[ti]

                    def chunk(sc, cnt, pi=pi, have_rid=have_rid,
                              have_w=have_w, tab=tab, sign=sign,
                              tbl_rows=tbl_rows, mid_hbm=mid_hbm,
                              rid_hbm=rid_hbm, w_hbm=w_hbm):
                        off = pl.multiple_of(sc * SC_CHUNK, 8)
                        cps = [pltpu.async_copy(
                            mid_hbm.at[pl.ds(off, SC_CHUNK)], stm_v, sem)]
                        if have_rid:
                            cps.append(pltpu.async_copy(
                                rid_hbm.at[pl.ds(off, SC_CHUNK)], str_v,
                                semr))
                        if have_w:
                            cps.append(pltpu.async_copy(
                                w_hbm.at[pl.ds(off, SC_CHUNK)], stw_v,
                                semw))
                        for cp in cps:
                            cp.wait()

                        def comp(g, cnt_v):
                            o = pl.multiple_of(g * 16, 8)
                            v = stm_v[pl.ds(o, 16)]
                            d = v - lo
                            du = lax.bitcast_convert_type(d, jnp.uint32)
                            m = du < jnp.uint32(R_ROWS)
                            mi = jnp.where(m, 1, 0)
                            cs = plsc.cumsum(mi)
                            cs_v[...] = cs
                            tot = plsc.load_gather(
                                cs_v, [jnp.full((16,), 15, jnp.int32)])
                            pos = cnt_v + cs - 1
                            plsc.store_scatter(cd_v, [pos], d, mask=m)
                            if have_rid:
                                r = str_v[pl.ds(o, 16)] + b * tbl_rows
                            else:
                                r = (off + o + iota) + b * tbl_rows
                            plsc.store_scatter(cr_v, [pos], r, mask=m)
                            if have_w:
                                plsc.store_scatter(
                                    cw_v, [pos], stw_v[pl.ds(o, 16)],
                                    mask=m)
                            return cnt_v + tot

                        cnt_v = lax.fori_loop(
                            0, SC_CHUNK // 16, comp,
                            jnp.full((16,), 0, jnp.int32) + cnt)
                        cnt2 = jnp.max(cnt_v)
                        ng = cnt2 >> 6
                        process(tab, ng, have_w, sign)
                        return carry_to_front(ng, cnt2)

                    cnt = lax.fori_loop(0, E_PAD // SC_CHUNK, chunk, 0)

                    for kk in range(G // 16):
                        pp = cnt + kk * 16 + iota
                        plsc.store_scatter(
                            cd_v, [pp], jnp.full((16,), R_ROWS, jnp.int32))
                        plsc.store_scatter(cr_v, [pp], (w * 16 + iota) & 63)
                        plsc.store_scatter(
                            cw_v, [pp], jnp.zeros((16,), jnp.float32))
                    process(tab, (cnt + (G - 1)) >> 6, have_w, sign)

                obase = pl.multiple_of(
                    (b * out_rows + (p * NW + w) * R_ROWS) * H, 8)
                pltpu.sync_copy(
                    acc_v.at[pl.ds(0, R_ROWS * H)],
                    out_hbm.at[pl.ds(obase, R_ROWS * H)])
                return 0

            lax.fori_loop(0, B, per_b, 0)
            return 0

        lax.fori_loop(0, n_pass, per_pass, 0)

    dummy_i = jnp.zeros((8,), jnp.int32)
    dummy_f = jnp.zeros((8,), jnp.float32)
    args = list(tabs)
    for (mask_arr, rid_arr, w_arr, _, _, _) in phases:
        args += [mask_arr,
                 rid_arr if rid_arr is not None else dummy_i,
                 w_arr if w_arr is not None else dummy_f]
    return k(*args)


def kernel(h, edge_index, b2_edges, b2_cells, We1, be1, We2, be2,
           Wc1, bc1, Wc2, bc2, Wn1, bn1, Wn2, bn2, Wg, bg):
    src = edge_index[0]
    dst = edge_index[1]

    deg_nodes = jnp.maximum(
        jnp.zeros((N,), jnp.float32).at[src].add(1.0).at[dst].add(1.0), 1.0)
    deg_cells = jnp.maximum(
        jnp.zeros((C,), jnp.float32).at[b2_cells].add(1.0), 1.0)
    deg_edges_c = jnp.maximum(
        jnp.zeros((E,), jnp.float32).at[b2_edges].add(1.0), 1.0)
    inv_dn = (1.0 / deg_nodes)[:, None]
    inv_dc = 1.0 / deg_cells
    wp = (1.0 / deg_edges_c)[b2_edges]
    pd = dst[b2_edges]
    ps = src[b2_edges]

    pad_iota = jnp.arange(E_PAD - E, dtype=jnp.int32)
    padN = lambda a: jnp.concatenate([a, pad_iota % N])
    padS = lambda a: jnp.concatenate(
        [a, jnp.full((E_PAD - E,), SENT, jnp.int32)])
    pad0i = lambda a: jnp.concatenate(
        [a, jnp.zeros((E_PAD - E,), jnp.int32)])
    pad0f = lambda a: jnp.concatenate(
        [a, jnp.zeros((E_PAD - E,), jnp.float32)])

    dst_p, src_p = padN(dst), padN(src)
    dst_m, src_m = padS(dst), padS(src)
    cells_m = padS(b2_cells)
    edges_r = pad0i(b2_edges)
    cells_r = pad0i(b2_cells)
    pd_m, ps_m = padS(pd), padS(ps)
    wp_p = pad0f(wp)

    b1r = lambda v: v[None, :]

    x_e = _edge_gather(h.reshape(B * N, H), dst_p, src_p)
    ones = jnp.ones((B * E_PAD, 1), jnp.float32)
    e_mlp = _fused_mlp(x_e.reshape(B * E_PAD, H), ones, We1, b1r(be1),
                       We2, b1r(be2))

    c_raw = _accumulate(
        [e_mlp], [(cells_m, edges_r, None, 1.0, 0, E_PAD)], 4, C_PAD,
        "cells")
    inv_dc_p = jnp.concatenate(
        [inv_dc, jnp.ones((C_PAD - C,), jnp.float32)])
    c_mlp = _fused_mlp(c_raw.reshape(B * C_PAD, H),
                       jnp.tile(inv_dc_p[:, None], (B, 1)),
                       Wc1, b1r(bc1), Wc2, b1r(bc2))

    hm = _accumulate(
        [e_mlp, c_mlp],
        [(dst_m, None, None, 1.0, 0, E_PAD),
         (src_m, None, None, -1.0, 0, E_PAD),
         (pd_m, cells_r, wp_p, 1.0, 1, C_PAD),
         (ps_m, cells_r, wp_p, -1.0, 1, C_PAD)],
        1, N_PAD, "nodes")
    hm_raw = hm.reshape(B, N_PAD, H)[:, :N, :]

    out = _fused_tail(hm_raw.reshape(B * N, H), jnp.tile(inv_dn, (B, 1)),
                      h.reshape(B * N, H), Wn1, b1r(bn1), Wn2, b1r(bn2),
                      Wg[:H], Wg[H:], b1r(bg))
    return out.reshape(B, N, H)


# speedup vs baseline: 5.4194x; 1.3672x over previous
import functools

import jax
import jax.numpy as jnp
from jax import lax
from jax.experimental import pallas as pl
from jax.experimental.pallas import tpu as pltpu
from jax.experimental.pallas import tpu_sc as plsc

H = 256
N = 10000
E = 160000
C = 40000
B = 4

NW = 32
EPW = 5120
E_PAD = EPW * NW
R_ROWS = 320
N_PAD = R_ROWS * NW
C_PAD = R_ROWS * NW * 4
SENT = 1 << 30

SC_CHUNK = 2048
G = 64
CBUF = SC_CHUNK + 2 * G


def _pick_blk(m):
    for b in (1024, 800, 512, 400, 320, 256, 160, 128, 80, 64, 8):
        if m % b == 0:
            return b
    raise ValueError(m)


def _mlp_body(x_ref, d_ref, w1_ref, b1_ref, w2_ref, b2_ref, o_ref):
    x = jnp.maximum(x_ref[...] * d_ref[...], 0.0)
    h1 = jnp.maximum(
        jax.lax.dot(x, w1_ref[...], preferred_element_type=jnp.float32)
        + b1_ref[...], 0.0)
    o_ref[...] = (
        jax.lax.dot(h1, w2_ref[...], preferred_element_type=jnp.float32)
        + b2_ref[...])


def _fused_mlp(x, inv_deg, w1, b1, w2, b2):
    m = x.shape[0]
    blk = _pick_blk(m)
    return pl.pallas_call(
        _mlp_body,
        grid=(m // blk,),
        in_specs=[
            pl.BlockSpec((blk, H), lambda i: (i, 0)),
            pl.BlockSpec((blk, 1), lambda i: (i, 0)),
            pl.BlockSpec((H, H), lambda i: (0, 0)),
            pl.BlockSpec((1, H), lambda i: (0, 0)),
            pl.BlockSpec((H, H), lambda i: (0, 0)),
            pl.BlockSpec((1, H), lambda i: (0, 0)),
        ],
        out_specs=pl.BlockSpec((blk, H), lambda i: (i, 0)),
        out_shape=jax.ShapeDtypeStruct((m, H), jnp.float32),
    )(x, inv_deg, w1, b1, w2, b2)


def _tail_body(x_ref, d_ref, hb_ref, w1_ref, b1_ref, w2_ref, b2_ref,
               wg1_ref, wg2_ref, bg_ref, o_ref):
    x = jnp.maximum(x_ref[...] * d_ref[...], 0.0)
    h1 = jnp.maximum(
        jax.lax.dot(x, w1_ref[...], preferred_element_type=jnp.float32)
        + b1_ref[...], 0.0)
    hm = (jax.lax.dot(h1, w2_ref[...], preferred_element_type=jnp.float32)
          + b2_ref[...])
    hb = hb_ref[...]
    z = jax.nn.sigmoid(
        jax.lax.dot(hb, wg1_ref[...], preferred_element_type=jnp.float32)
        + jax.lax.dot(hm, wg2_ref[...], preferred_element_type=jnp.float32)
        + bg_ref[...])
    o_ref[...] = hb + z * hm


def _fused_tail(x, inv_deg, hb, w1, b1, w2, b2, wg1, wg2, bg):
    m = x.shape[0]
    blk = _pick_blk(m)
    full = lambda i: (0, 0)
    return pl.pallas_call(
        _tail_body,
        grid=(m // blk,),
        in_specs=[
            pl.BlockSpec((blk, H), lambda i: (i, 0)),
            pl.BlockSpec((blk, 1), lambda i: (i, 0)),
            pl.BlockSpec((blk, H), lambda i: (i, 0)),
            pl.BlockSpec((H, H), full),
            pl.BlockSpec((1, H), full),
            pl.BlockSpec((H, H), full),
            pl.BlockSpec((1, H), full),
            pl.BlockSpec((H, H), full),
            pl.BlockSpec((H, H), full),
            pl.BlockSpec((1, H), full),
        ],
        out_specs=pl.BlockSpec((blk, H), lambda i: (i, 0)),
        out_shape=jax.ShapeDtypeStruct((m, H), jnp.float32),
    )(x, inv_deg, hb, w1, b1, w2, b2, wg1, wg2, bg)


def _sc_mesh():
    return plsc.VectorSubcoreMesh(core_axis_name="c", subcore_axis_name="s")


_SC_PARAMS = dict(
    compiler_params=pltpu.CompilerParams(needs_layout_passes=False))


def _wid():
    return lax.axis_index("s") * 2 + lax.axis_index("c")


def _edge_gather(h_flat, didx, sidx):
    KC = 128

    @functools.partial(
        pl.kernel,
        out_type=jax.ShapeDtypeStruct((B * E_PAD, H), jnp.float32),
        mesh=_sc_mesh(),
        scratch_types=[
            pltpu.VMEM((KC,), jnp.int32),
            pltpu.VMEM((KC,), jnp.int32),
            pltpu.VMEM((KC,), jnp.int32),
            pltpu.VMEM((KC,), jnp.int32),
            pltpu.VMEM((KC, H), jnp.float32),
            pltpu.VMEM((KC, H), jnp.float32),
            pltpu.SemaphoreType.DMA,
            pltpu.SemaphoreType.DMA,
        ],
        **_SC_PARAMS,
    )
    def k(h_hbm, d_hbm, s_hbm, out_hbm, di_v, si_v, ib1_v, ib2_v, rd_v,
          rs_v, sem1, sem2):
        w = _wid()
        ebase = w * EPW

        def chunk(ch, _):
            off = pl.multiple_of(ebase + ch * KC, 8)
            pltpu.sync_copy(d_hbm.at[pl.ds(off, KC)], di_v)
            pltpu.sync_copy(s_hbm.at[pl.ds(off, KC)], si_v)

            def per_b(b, _):
                for kk in range(KC // 16):
                    ib1_v[pl.ds(kk * 16, 16)] = (
                        di_v[pl.ds(kk * 16, 16)] + b * N)
                    ib2_v[pl.ds(kk * 16, 16)] = (
                        si_v[pl.ds(kk * 16, 16)] + b * N)
                cp1 = pltpu.async_copy(h_hbm.at[ib1_v], rd_v, sem1)
                cp2 = pltpu.async_copy(h_hbm.at[ib2_v], rs_v, sem2)
                cp1.wait()
                cp2.wait()

                def row(rr, _):
                    for j in range(H // 16):
                        d = rd_v[rr, pl.ds(j * 16, 16)]
                        s = rs_v[rr, pl.ds(j * 16, 16)]
                        rd_v[rr, pl.ds(j * 16, 16)] = jnp.maximum(
                            (d - s) * 0.5, 0.0)
                    return 0

                lax.fori_loop(0, KC, row, 0)
                orow = b * E_PAD + ebase + ch * KC
                pltpu.sync_copy(rd_v, out_hbm.at[pl.ds(orow, KC)])
                return 0

            lax.fori_loop(0, B, per_b, 0)
            return 0

        lax.fori_loop(0, EPW // KC, chunk, 0)

    return k(h_flat, didx, sidx)


def _accumulate(tabs, phases, n_pass, out_rows, tag):
    n_tab = len(tabs)
    n_ph = len(phases)
    n_stream = n_ph * n_pass
    SLEN = E_PAD + SC_CHUNK
    scr = [
        pltpu.VMEM((SC_CHUNK,), jnp.int32),
        pltpu.VMEM((SC_CHUNK,), jnp.int32),
        pltpu.VMEM((SC_CHUNK,), jnp.float32),
        pltpu.VMEM((CBUF,), jnp.int32),
        pltpu.VMEM((CBUF,), jnp.int32),
        pltpu.VMEM((CBUF,), jnp.float32),
        pltpu.VMEM((16,), jnp.int32),
        pltpu.VMEM((G,), jnp.int32),
        pltpu.VMEM((G, H), jnp.float32),
        pltpu.VMEM(((R_ROWS + 1) * H,), jnp.float32),
        pltpu.SemaphoreType.DMA,
        pltpu.SemaphoreType.DMA,
        pltpu.SemaphoreType.DMA,
        pltpu.SemaphoreType.DMA,
    ]

    out_types = [
        jax.ShapeDtypeStruct((B * out_rows * H,), jnp.float32),
        jax.ShapeDtypeStruct((n_stream * NW * SLEN,), jnp.int32),
        jax.ShapeDtypeStruct((n_stream * NW * SLEN,), jnp.int32),
        jax.ShapeDtypeStruct((n_stream * NW * SLEN,), jnp.float32),
        jax.ShapeDtypeStruct((n_stream * NW * 16,), jnp.int32),
    ]

    @functools.partial(
        pl.kernel,
        out_type=out_types,
        mesh=_sc_mesh(),
        scratch_types=scr,
        name=f"sc_accum_{tag}",
        **_SC_PARAMS,
    )
    def k(*refs):
        tab_hbm = refs[:n_tab]
        ph_hbm = refs[n_tab:n_tab + 3 * n_ph]
        out_hbm, scd_hbm, scr_hbm, scw_hbm, cnts_hbm = (
            refs[n_tab + 3 * n_ph:n_tab + 3 * n_ph + 5])
        (stm_v, str_v, stw_v, cd_v, cr_v, cw_v, cs_v, gi_v, rows_v,
         acc_v, sg, sem, semr, semw) = refs[n_tab + 3 * n_ph + 5:]
        w = _wid()
        iota = lax.iota(jnp.int32, 16)

        for p in range(n_pass):
            lo = (p * NW + w) * R_ROWS
            for pi, (_, _, _, sign, ti, tbl_rows) in enumerate(phases):
                si = p * n_ph + pi
                sbase = (si * NW + w) * SLEN
                have_rid = phases[pi][1] is not None
                have_w = phases[pi][2] is not None
                mid_hbm = ph_hbm[3 * pi]
                rid_hbm = ph_hbm[3 * pi + 1]
                w_hbm = ph_hbm[3 * pi + 2]

                def chunk(sc, carry, have_rid=have_rid, have_w=have_w,
                          mid_hbm=mid_hbm, rid_hbm=rid_hbm, w_hbm=w_hbm,
                          lo=lo, sbase=sbase):
                    cnt, blk = carry
                    off = pl.multiple_of(sc * SC_CHUNK, 8)
                    cps = [pltpu.async_copy(
                        mid_hbm.at[pl.ds(off, SC_CHUNK)], stm_v, sem)]
                    if have_rid:
                        cps.append(pltpu.async_copy(
                            rid_hbm.at[pl.ds(off, SC_CHUNK)], str_v, semr))
                    if have_w:
                        cps.append(pltpu.async_copy(
                            w_hbm.at[pl.ds(off, SC_CHUNK)], stw_v, semw))
                    for cp in cps:
                        cp.wait()

                    def comp(g, cnt_v):
                        o = pl.multiple_of(g * 16, 8)
                        v = stm_v[pl.ds(o, 16)]
                        d = v - lo
                        du = lax.bitcast_convert_type(d, jnp.uint32)
                        m = du < jnp.uint32(R_ROWS)
                        mi = jnp.where(m, 1, 0)
                        cs = plsc.cumsum(mi)
                        cs_v[...] = cs
                        tot = plsc.load_gather(
                            cs_v, [jnp.full((16,), 15, jnp.int32)])
                        pos = cnt_v + cs - 1
                        plsc.store_scatter(cd_v, [pos], d, mask=m)
                        if have_rid:
                            r = str_v[pl.ds(o, 16)]
                        else:
                            r = off + o + iota
                        plsc.store_scatter(cr_v, [pos], r, mask=m)
                        if have_w:
                            plsc.store_scatter(
                                cw_v, [pos], stw_v[pl.ds(o, 16)], mask=m)
                        return cnt_v + tot

                    cnt_v = lax.fori_loop(
                        0, SC_CHUNK // 16, comp,
                        jnp.full((16,), 0, jnp.int32) + cnt)
                    cnt2 = jnp.max(cnt_v)

                    def wr(args, have_w=have_w, sbase=sbase):
                        cnt3, blk3 = args
                        ob = pl.multiple_of(sbase + blk3 * SC_CHUNK, 8)
                        pltpu.sync_copy(cd_v.at[pl.ds(0, SC_CHUNK)],
                                        scd_hbm.at[pl.ds(ob, SC_CHUNK)])
                        pltpu.sync_copy(cr_v.at[pl.ds(0, SC_CHUNK)],
                                        scr_hbm.at[pl.ds(ob, SC_CHUNK)])
                        if have_w:
                            pltpu.sync_copy(cw_v.at[pl.ds(0, SC_CHUNK)],
                                            scw_hbm.at[pl.ds(ob, SC_CHUNK)])
                        for kk in range(G // 16):
                            sp = SC_CHUNK + kk * 16 + iota
                            dp = kk * 16 + iota
                            plsc.store_scatter(
                                cd_v, [dp], plsc.load_gather(cd_v, [sp]))
                            plsc.store_scatter(
                                cr_v, [dp], plsc.load_gather(cr_v, [sp]))
                            if have_w:
                                plsc.store_scatter(
                                    cw_v, [dp],
                                    plsc.load_gather(cw_v, [sp]))
                        return (cnt3 - SC_CHUNK, blk3 + 1)

                    return lax.cond(cnt2 >= SC_CHUNK, wr, lambda a: a,
                                    (cnt2, blk))

                cnt, blk = lax.fori_loop(0, E_PAD // SC_CHUNK, chunk,
                                         (0, 0))

                for kk in range(G // 16):
                    pp = cnt + kk * 16 + iota
                    plsc.store_scatter(
                        cd_v, [pp], jnp.full((16,), R_ROWS, jnp.int32))
                    plsc.store_scatter(
                        cr_v, [pp], (w * 16 + iota) & jnp.int32(1023))
                    if have_w:
                        plsc.store_scatter(
                            cw_v, [pp], jnp.zeros((16,), jnp.float32))
                ob = pl.multiple_of(sbase + blk * SC_CHUNK, 8)
                pltpu.sync_copy(cd_v.at[pl.ds(0, SC_CHUNK)],
                                scd_hbm.at[pl.ds(ob, SC_CHUNK)])
                pltpu.sync_copy(cr_v.at[pl.ds(0, SC_CHUNK)],
                                scr_hbm.at[pl.ds(ob, SC_CHUNK)])
                if have_w:
                    pltpu.sync_copy(cw_v.at[pl.ds(0, SC_CHUNK)],
                                    scw_hbm.at[pl.ds(ob, SC_CHUNK)])
                ngt = blk * (SC_CHUNK // G) + ((cnt + (G - 1)) >> 6)
                cs_v[...] = jnp.full((16,), 0, jnp.int32) + ngt
                pltpu.sync_copy(
                    cs_v, cnts_hbm.at[pl.ds((si * NW + w) * 16, 16)])

        for p in range(n_pass):
            def per_b(b, _, p=p):
                def z(i, _):
                    acc_v[pl.ds(pl.multiple_of(i * 16, 8), 16)] = (
                        jnp.zeros((16,), jnp.float32))
                    return 0
                lax.fori_loop(0, (R_ROWS + 1) * H // 16, z, 0)

                for pi, (_, _, _, sign, ti, tbl_rows) in enumerate(phases):
                    si = p * n_ph + pi
                    sbase = (si * NW + w) * SLEN
                    have_w = phases[pi][2] is not None
                    tab = tab_hbm[ti]

                    pltpu.sync_copy(
                        cnts_hbm.at[pl.ds((si * NW + w) * 16, 16)], cs_v)
                    ngt = jnp.max(cs_v[...])
                    nblk = (ngt + (SC_CHUNK // G - 1)) >> 5

                    def blkf(bk, _, have_w=have_w, tab=tab, sign=sign,
                             tbl_rows=tbl_rows, sbase=sbase, ngt=ngt,
                             b=b):
                        ob = pl.multiple_of(sbase + bk * SC_CHUNK, 8)
                        cps = [pltpu.async_copy(
                            scd_hbm.at[pl.ds(ob, SC_CHUNK)],
                            cd_v.at[pl.ds(0, SC_CHUNK)], sem),
                            pltpu.async_copy(
                                scr_hbm.at[pl.ds(ob, SC_CHUNK)],
                                cr_v.at[pl.ds(0, SC_CHUNK)], semr)]
                        if have_w:
                            cps.append(pltpu.async_copy(
                                scw_hbm.at[pl.ds(ob, SC_CHUNK)],
                                cw_v.at[pl.ds(0, SC_CHUNK)], semw))
                        for cp in cps:
                            cp.wait()
                        ning = jnp.minimum(SC_CHUNK // G,
                                           ngt - bk * (SC_CHUNK // G))

                        def grp(g, _):
                            gb = g * G
                            for kk in range(G // 16):
                                gi_v[pl.ds(kk * 16, 16)] = plsc.load_gather(
                                    cr_v,
                                    [gb + kk * 16 + iota]) + b * tbl_rows
                            pltpu.async_copy(tab.at[gi_v], rows_v,
                                             sg).wait()

                            def row(rr, _):
                                d16 = plsc.load_gather(
                                    cd_v,
                                    [jnp.full((16,), gb + rr, jnp.int32)])
                                dbase = d16 * H + iota
                                if have_w:
                                    w16 = plsc.load_gather(
                                        cw_v, [jnp.full((16,), gb + rr,
                                                        jnp.int32)])
                                    scale = w16 * sign
                                else:
                                    scale = None
                                for j in range(H // 16):
                                    seg = rows_v[rr, pl.ds(j * 16, 16)]
                                    if scale is not None:
                                        seg = seg * scale
                                    elif sign != 1.0:
                                        seg = seg * sign
                                    plsc.addupdate_scatter(
                                        acc_v, [dbase + j * 16], seg)
                                return 0

                            lax.fori_loop(0, G, row, 0)
                            return 0

                        lax.fori_loop(0, ning, grp, 0)
                        return 0

                    lax.fori_loop(0, nblk, blkf, 0)

                obase = pl.multiple_of(
                    (b * out_rows + (p * NW + w) * R_ROWS) * H, 8)
                pltpu.sync_copy(
                    acc_v.at[pl.ds(0, R_ROWS * H)],
                    out_hbm.at[pl.ds(obase, R_ROWS * H)])
                return 0

            lax.fori_loop(0, B, per_b, 0)

    dummy_i = jnp.zeros((8,), jnp.int32)
    dummy_f = jnp.zeros((8,), jnp.float32)
    args = list(tabs)
    for (mask_arr, rid_arr, w_arr, _, _, _) in phases:
        args += [mask_arr,
                 rid_arr if rid_arr is not None else dummy_i,
                 w_arr if w_arr is not None else dummy_f]
    return k(*args)[0]


def kernel(h, edge_index, b2_edges, b2_cells, We1, be1, We2, be2,
           Wc1, bc1, Wc2, bc2, Wn1, bn1, Wn2, bn2, Wg, bg):
    src = edge_index[0]
    dst = edge_index[1]

    deg_nodes = jnp.maximum(
        jnp.zeros((N,), jnp.float32).at[src].add(1.0).at[dst].add(1.0), 1.0)
    deg_cells = jnp.maximum(
        jnp.zeros((C,), jnp.float32).at[b2_cells].add(1.0), 1.0)
    deg_edges_c = jnp.maximum(
        jnp.zeros((E,), jnp.float32).at[b2_edges].add(1.0), 1.0)
    inv_dn = (1.0 / deg_nodes)[:, None]
    inv_dc = 1.0 / deg_cells
    wp = (1.0 / deg_edges_c)[b2_edges]
    pd = dst[b2_edges]
    ps = src[b2_edges]

    pad_iota = jnp.arange(E_PAD - E, dtype=jnp.int32)
    padN = lambda a: jnp.concatenate([a, pad_iota % N])
    padS = lambda a: jnp.concatenate(
        [a, jnp.full((E_PAD - E,), SENT, jnp.int32)])
    pad0i = lambda a: jnp.concatenate(
        [a, jnp.zeros((E_PAD - E,), jnp.int32)])
    pad0f = lambda a: jnp.concatenate(
        [a, jnp.zeros((E_PAD - E,), jnp.float32)])

    dst_p, src_p = padN(dst), padN(src)
    dst_m, src_m = padS(dst), padS(src)
    cells_m = padS(b2_cells)
    edges_r = pad0i(b2_edges)
    cells_r = pad0i(b2_cells)
    pd_m, ps_m = padS(pd), padS(ps)
    wp_p = pad0f(wp)

    b1r = lambda v: v[None, :]

    x_e = _edge_gather(h.reshape(B * N, H), dst_p, src_p)
    ones = jnp.ones((B * E_PAD, 1), jnp.float32)
    e_mlp = _fused_mlp(x_e.reshape(B * E_PAD, H), ones, We1, b1r(be1),
                       We2, b1r(be2))

    c_raw = _accumulate(
        [e_mlp], [(cells_m, edges_r, None, 1.0, 0, E_PAD)], 4, C_PAD,
        "cells")
    inv_dc_p = jnp.concatenate(
        [inv_dc, jnp.ones((C_PAD - C,), jnp.float32)])
    c_mlp = _fused_mlp(c_raw.reshape(B * C_PAD, H),
                       jnp.tile(inv_dc_p[:, None], (B, 1)),
                       Wc1, b1r(bc1), Wc2, b1r(bc2))

    hm = _accumulate(
        [e_mlp, c_mlp],
        [(dst_m, None, None, 1.0, 0, E_PAD),
         (src_m, None, None, -1.0, 0, E_PAD),
         (pd_m, cells_r, wp_p, 1.0, 1, C_PAD),
         (ps_m, cells_r, wp_p, -1.0, 1, C_PAD)],
        1, N_PAD, "nodes")
    hm_raw = hm.reshape(B, N_PAD, H)[:, :N, :]

    out = _fused_tail(hm_raw.reshape(B * N, H), jnp.tile(inv_dn, (B, 1)),
                      h.reshape(B * N, H), Wn1, b1r(bn1), Wn2, b1r(bn2),
                      Wg[:H], Wg[H:], b1r(bg))
    return out.reshape(B, N, H)


# speedup vs baseline: 5.5588x; 1.0257x over previous
import functools

import jax
import jax.numpy as jnp
from jax import lax
from jax.experimental import pallas as pl
from jax.experimental.pallas import tpu as pltpu
from jax.experimental.pallas import tpu_sc as plsc

H = 256
N = 10000
E = 160000
C = 40000
B = 4

NW = 32
EPW = 5120
E_PAD = EPW * NW
R_ROWS = 316
N_PAD = R_ROWS * NW
C_PAD = R_ROWS * NW * 4
SENT = 1 << 30

SC_CHUNK = 2048
G = 32
LG = 5
GPB = SC_CHUNK // G
LGPB = 6
CBUF = 2 * SC_CHUNK


def _pick_blk(m):
    for b in (1024, 800, 512, 400, 320, 256, 160, 128, 80, 64, 8):
        if m % b == 0:
            return b
    raise ValueError(m)


def _mlp_body(x_ref, d_ref, w1_ref, b1_ref, w2_ref, b2_ref, o_ref):
    x = jnp.maximum(x_ref[...] * d_ref[...], 0.0)
    h1 = jnp.maximum(
        jax.lax.dot(x, w1_ref[...], preferred_element_type=jnp.float32)
        + b1_ref[...], 0.0)
    o_ref[...] = (
        jax.lax.dot(h1, w2_ref[...], preferred_element_type=jnp.float32)
        + b2_ref[...])


def _fused_mlp(x, inv_deg, w1, b1, w2, b2):
    m = x.shape[0]
    blk = _pick_blk(m)
    return pl.pallas_call(
        _mlp_body,
        grid=(m // blk,),
        in_specs=[
            pl.BlockSpec((blk, H), lambda i: (i, 0)),
            pl.BlockSpec((blk, 1), lambda i: (i, 0)),
            pl.BlockSpec((H, H), lambda i: (0, 0)),
            pl.BlockSpec((1, H), lambda i: (0, 0)),
            pl.BlockSpec((H, H), lambda i: (0, 0)),
            pl.BlockSpec((1, H), lambda i: (0, 0)),
        ],
        out_specs=pl.BlockSpec((blk, H), lambda i: (i, 0)),
        out_shape=jax.ShapeDtypeStruct((m, H), jnp.float32),
    )(x, inv_deg, w1, b1, w2, b2)


def _tail_body(x_ref, d_ref, hb_ref, w1_ref, b1_ref, w2_ref, b2_ref,
               wg1_ref, wg2_ref, bg_ref, o_ref):
    x = jnp.maximum(x_ref[...] * d_ref[...], 0.0)
    h1 = jnp.maximum(
        jax.lax.dot(x, w1_ref[...], preferred_element_type=jnp.float32)
        + b1_ref[...], 0.0)
    hm = (jax.lax.dot(h1, w2_ref[...], preferred_element_type=jnp.float32)
          + b2_ref[...])
    hb = hb_ref[...]
    z = jax.nn.sigmoid(
        jax.lax.dot(hb, wg1_ref[...], preferred_element_type=jnp.float32)
        + jax.lax.dot(hm, wg2_ref[...], preferred_element_type=jnp.float32)
        + bg_ref[...])
    o_ref[...] = hb + z * hm


def _fused_tail(x, inv_deg, hb, w1, b1, w2, b2, wg1, wg2, bg):
    m = x.shape[0]
    blk = _pick_blk(m)
    full = lambda i: (0, 0)
    return pl.pallas_call(
        _tail_body,
        grid=(m // blk,),
        in_specs=[
            pl.BlockSpec((blk, H), lambda i: (i, 0)),
            pl.BlockSpec((blk, 1), lambda i: (i, 0)),
            pl.BlockSpec((blk, H), lambda i: (i, 0)),
            pl.BlockSpec((H, H), full),
            pl.BlockSpec((1, H), full),
            pl.BlockSpec((H, H), full),
            pl.BlockSpec((1, H), full),
            pl.BlockSpec((H, H), full),
            pl.BlockSpec((H, H), full),
            pl.BlockSpec((1, H), full),
        ],
        out_specs=pl.BlockSpec((blk, H), lambda i: (i, 0)),
        out_shape=jax.ShapeDtypeStruct((m, H), jnp.float32),
    )(x, inv_deg, hb, w1, b1, w2, b2, wg1, wg2, bg)


def _sc_mesh():
    return plsc.VectorSubcoreMesh(core_axis_name="c", subcore_axis_name="s")


_SC_PARAMS = dict(
    compiler_params=pltpu.CompilerParams(needs_layout_passes=False))


def _wid():
    return lax.axis_index("s") * 2 + lax.axis_index("c")


def _edge_gather(h_flat, didx, sidx):
    KC = 128

    @functools.partial(
        pl.kernel,
        out_type=jax.ShapeDtypeStruct((B * E_PAD, H), jnp.float32),
        mesh=_sc_mesh(),
        scratch_types=[
            pltpu.VMEM((KC,), jnp.int32),
            pltpu.VMEM((KC,), jnp.int32),
            pltpu.VMEM((KC,), jnp.int32),
            pltpu.VMEM((KC,), jnp.int32),
            pltpu.VMEM((KC, H), jnp.float32),
            pltpu.VMEM((KC, H), jnp.float32),
            pltpu.SemaphoreType.DMA,
            pltpu.SemaphoreType.DMA,
        ],
        **_SC_PARAMS,
    )
    def k(h_hbm, d_hbm, s_hbm, out_hbm, di_v, si_v, ib1_v, ib2_v, rd_v,
          rs_v, sem1, sem2):
        w = _wid()
        ebase = w * EPW

        def chunk(ch, _):
            off = pl.multiple_of(ebase + ch * KC, 8)
            pltpu.sync_copy(d_hbm.at[pl.ds(off, KC)], di_v)
            pltpu.sync_copy(s_hbm.at[pl.ds(off, KC)], si_v)

            def per_b(b, _):
                for kk in range(KC // 16):
                    ib1_v[pl.ds(kk * 16, 16)] = (
                        di_v[pl.ds(kk * 16, 16)] + b * N)
                    ib2_v[pl.ds(kk * 16, 16)] = (
                        si_v[pl.ds(kk * 16, 16)] + b * N)
                cp1 = pltpu.async_copy(h_hbm.at[ib1_v], rd_v, sem1)
                cp2 = pltpu.async_copy(h_hbm.at[ib2_v], rs_v, sem2)
                cp1.wait()
                cp2.wait()

                def row(rr, _):
                    for j in range(H // 16):
                        d = rd_v[rr, pl.ds(j * 16, 16)]
                        s = rs_v[rr, pl.ds(j * 16, 16)]
                        rd_v[rr, pl.ds(j * 16, 16)] = jnp.maximum(
                            (d - s) * 0.5, 0.0)
                    return 0

                lax.fori_loop(0, KC, row, 0)
                orow = b * E_PAD + ebase + ch * KC
                pltpu.sync_copy(rd_v, out_hbm.at[pl.ds(orow, KC)])
                return 0

            lax.fori_loop(0, B, per_b, 0)
            return 0

        lax.fori_loop(0, EPW // KC, chunk, 0)

    return k(h_flat, didx, sidx)


def _accumulate(tabs, phases, n_pass, out_rows, tag):
    n_tab = len(tabs)
    n_ph = len(phases)
    n_stream = n_ph * n_pass
    SLEN = E_PAD + 3 * SC_CHUNK
    scr = [
        pltpu.VMEM((SC_CHUNK,), jnp.int32),
        pltpu.VMEM((SC_CHUNK,), jnp.int32),
        pltpu.VMEM((SC_CHUNK,), jnp.float32),
        pltpu.VMEM((CBUF,), jnp.int32),
        pltpu.VMEM((CBUF,), jnp.int32),
        pltpu.VMEM((CBUF,), jnp.float32),
        pltpu.VMEM((16,), jnp.int32),
        pltpu.VMEM((G,), jnp.int32),
        pltpu.VMEM((G, H), jnp.float32),
        pltpu.VMEM((G,), jnp.int32),
        pltpu.VMEM((G, H), jnp.float32),
        pltpu.VMEM(((R_ROWS + 1) * H,), jnp.float32),
        pltpu.SemaphoreType.DMA,
        pltpu.SemaphoreType.DMA,
        pltpu.SemaphoreType.DMA,
        pltpu.SemaphoreType.DMA,
        pltpu.SemaphoreType.DMA,
    ]

    out_types = [
        jax.ShapeDtypeStruct((B * out_rows * H,), jnp.float32),
        jax.ShapeDtypeStruct((n_stream * NW * SLEN,), jnp.int32),
        jax.ShapeDtypeStruct((n_stream * NW * SLEN,), jnp.int32),
        jax.ShapeDtypeStruct((n_stream * NW * SLEN,), jnp.float32),
        jax.ShapeDtypeStruct((n_stream * NW * 16,), jnp.int32),
    ]

    @functools.partial(
        pl.kernel,
        out_type=out_types,
        mesh=_sc_mesh(),
        scratch_types=scr,
        name=f"sc_accum_{tag}",
        **_SC_PARAMS,
    )
    def k(*refs):
        tab_hbm = refs[:n_tab]
        ph_hbm = refs[n_tab:n_tab + 3 * n_ph]
        out_hbm, scd_hbm, scr_hbm, scw_hbm, cnts_hbm = (
            refs[n_tab + 3 * n_ph:n_tab + 3 * n_ph + 5])
        (stm_v, str_v, stw_v, cd_v, cr_v, cw_v, cs_v, gi_v, rows_v,
         gi2_v, rows2_v, acc_v, sg, sg2, sem, semr,
         semw) = refs[n_tab + 3 * n_ph + 5:]
        w = _wid()
        iota = lax.iota(jnp.int32, 16)

        for p in range(n_pass):
            lo = (p * NW + w) * R_ROWS
            for pi, (_, _, _, sign, ti, tbl_rows) in enumerate(phases):
                si = p * n_ph + pi
                sbase = (si * NW + w) * SLEN
                have_rid = phases[pi][1] is not None
                have_w = phases[pi][2] is not None
                mid_hbm = ph_hbm[3 * pi]
                rid_hbm = ph_hbm[3 * pi + 1]
                w_hbm = ph_hbm[3 * pi + 2]

                def chunk(sc, carry, have_rid=have_rid, have_w=have_w,
                          mid_hbm=mid_hbm, rid_hbm=rid_hbm, w_hbm=w_hbm,
                          lo=lo, sbase=sbase):
                    cnt, blk = carry
                    off = pl.multiple_of(sc * SC_CHUNK, 8)
                    cps = [pltpu.async_copy(
                        mid_hbm.at[pl.ds(off, SC_CHUNK)], stm_v, sem)]
                    if have_rid:
                        cps.append(pltpu.async_copy(
                            rid_hbm.at[pl.ds(off, SC_CHUNK)], str_v, semr))
                    if have_w:
                        cps.append(pltpu.async_copy(
                            w_hbm.at[pl.ds(off, SC_CHUNK)], stw_v, semw))
                    for cp in cps:
                        cp.wait()

                    def comp(g, cnt_v):
                        o = pl.multiple_of(g * 16, 8)
                        v = stm_v[pl.ds(o, 16)]
                        d = v - lo
                        du = lax.bitcast_convert_type(d, jnp.uint32)
                        m = du < jnp.uint32(R_ROWS)
                        mi = jnp.where(m, 1, 0)
                        cs = plsc.cumsum(mi)
                        cs_v[...] = cs
                        tot = plsc.load_gather(
                            cs_v, [jnp.full((16,), 15, jnp.int32)])
                        pos = cnt_v + cs - 1
                        plsc.store_scatter(cd_v, [pos], d, mask=m)
                        if have_rid:
                            r = str_v[pl.ds(o, 16)]
                        else:
                            r = off + o + iota
                        plsc.store_scatter(cr_v, [pos], r, mask=m)
                        if have_w:
                            plsc.store_scatter(
                                cw_v, [pos], stw_v[pl.ds(o, 16)], mask=m)
                        return cnt_v + tot

                    cnt_v = lax.fori_loop(
                        0, SC_CHUNK // 16, comp,
                        jnp.full((16,), 0, jnp.int32) + cnt)
                    cnt2 = jnp.max(cnt_v)

                    def wr(args, have_w=have_w, sbase=sbase):
                        cnt3, blk3 = args
                        ob = pl.multiple_of(sbase + blk3 * SC_CHUNK, 8)
                        pltpu.sync_copy(cd_v.at[pl.ds(0, SC_CHUNK)],
                                        scd_hbm.at[pl.ds(ob, SC_CHUNK)])
                        pltpu.sync_copy(cr_v.at[pl.ds(0, SC_CHUNK)],
                                        scr_hbm.at[pl.ds(ob, SC_CHUNK)])
                        if have_w:
                            pltpu.sync_copy(cw_v.at[pl.ds(0, SC_CHUNK)],
                                            scw_hbm.at[pl.ds(ob, SC_CHUNK)])
                        for kk in range(G // 16):
                            sp = SC_CHUNK + kk * 16 + iota
                            dp = kk * 16 + iota
                            plsc.store_scatter(
                                cd_v, [dp], plsc.load_gather(cd_v, [sp]))
                            plsc.store_scatter(
                                cr_v, [dp], plsc.load_gather(cr_v, [sp]))
                            if have_w:
                                plsc.store_scatter(
                                    cw_v, [dp],
                                    plsc.load_gather(cw_v, [sp]))
                        return (cnt3 - SC_CHUNK, blk3 + 1)

                    return lax.cond(cnt2 >= SC_CHUNK, wr, lambda a: a,
                                    (cnt2, blk))

                cnt, blk = lax.fori_loop(0, E_PAD // SC_CHUNK, chunk,
                                         (0, 0))

                for kk in range(2 * G // 16):
                    pp = cnt + kk * 16 + iota
                    plsc.store_scatter(
                        cd_v, [pp], jnp.full((16,), R_ROWS, jnp.int32))
                    plsc.store_scatter(
                        cr_v, [pp], (w * 16 + iota) & jnp.int32(1023))
                    if have_w:
                        plsc.store_scatter(
                            cw_v, [pp], jnp.zeros((16,), jnp.float32))
                ob = pl.multiple_of(sbase + blk * SC_CHUNK, 8)
                pltpu.sync_copy(cd_v.at[pl.ds(0, SC_CHUNK)],
                                scd_hbm.at[pl.ds(ob, SC_CHUNK)])
                pltpu.sync_copy(cd_v.at[pl.ds(SC_CHUNK, SC_CHUNK)],
                                scd_hbm.at[pl.ds(ob + SC_CHUNK, SC_CHUNK)])
                pltpu.sync_copy(cr_v.at[pl.ds(0, SC_CHUNK)],
                                scr_hbm.at[pl.ds(ob, SC_CHUNK)])
                pltpu.sync_copy(cr_v.at[pl.ds(SC_CHUNK, SC_CHUNK)],
                                scr_hbm.at[pl.ds(ob + SC_CHUNK, SC_CHUNK)])
                if have_w:
                    pltpu.sync_copy(cw_v.at[pl.ds(0, SC_CHUNK)],
                                    scw_hbm.at[pl.ds(ob, SC_CHUNK)])
                    pltpu.sync_copy(
                        cw_v.at[pl.ds(SC_CHUNK, SC_CHUNK)],
                        scw_hbm.at[pl.ds(ob + SC_CHUNK, SC_CHUNK)])
                ngt = (blk * GPB
                       + (((cnt + (2 * G - 1)) >> (LG + 1)) << 1))
                cs_v[...] = jnp.full((16,), 0, jnp.int32) + ngt
                pltpu.sync_copy(
                    cs_v, cnts_hbm.at[pl.ds((si * NW + w) * 16, 16)])

        for p in range(n_pass):
            def per_b(b, _, p=p):
                def z(i, _):
                    acc_v[pl.ds(pl.multiple_of(i * 16, 8), 16)] = (
                        jnp.zeros((16,), jnp.float32))
                    return 0
                lax.fori_loop(0, (R_ROWS + 1) * H // 16, z, 0)

                for pi, (_, _, _, sign, ti, tbl_rows) in enumerate(phases):
                    si = p * n_ph + pi
                    sbase = (si * NW + w) * SLEN
                    have_w = phases[pi][2] is not None
                    tab = tab_hbm[ti]

                    pltpu.sync_copy(
                        cnts_hbm.at[pl.ds((si * NW + w) * 16, 16)], cs_v)
                    ngt = jnp.max(cs_v[...])
                    nblk = (ngt + (GPB - 1)) >> LGPB

                    def blkf(bk, _, have_w=have_w, tab=tab, sign=sign,
                             tbl_rows=tbl_rows, sbase=sbase, ngt=ngt,
                             b=b):
                        ob = pl.multiple_of(sbase + bk * SC_CHUNK, 8)
                        cps = [pltpu.async_copy(
                            scd_hbm.at[pl.ds(ob, SC_CHUNK)],
                            cd_v.at[pl.ds(0, SC_CHUNK)], sem),
                            pltpu.async_copy(
                                scr_hbm.at[pl.ds(ob, SC_CHUNK)],
                                cr_v.at[pl.ds(0, SC_CHUNK)], semr)]
                        if have_w:
                            cps.append(pltpu.async_copy(
                                scw_hbm.at[pl.ds(ob, SC_CHUNK)],
                                cw_v.at[pl.ds(0, SC_CHUNK)], semw))
                        for cp in cps:
                            cp.wait()
                        ning = jnp.minimum(GPB, ngt - bk * GPB)

                        def accum(gb, rows, _have_w=have_w, _sign=sign):
                            def row(rr, _):
                                d16 = plsc.load_gather(
                                    cd_v,
                                    [jnp.full((16,), gb + rr, jnp.int32)])
                                dbase = d16 * H + iota
                                if _have_w:
                                    w16 = plsc.load_gather(
                                        cw_v, [jnp.full((16,), gb + rr,
                                                        jnp.int32)])
                                    scale = w16 * _sign
                                else:
                                    scale = None
                                for j in range(H // 16):
                                    seg = rows[rr, pl.ds(j * 16, 16)]
                                    if scale is not None:
                                        seg = seg * scale
                                    elif _sign != 1.0:
                                        seg = seg * _sign
                                    plsc.addupdate_scatter(
                                        acc_v, [dbase + j * 16], seg)
                                return 0

                            lax.fori_loop(0, G, row, 0)

                        def grp2(i, _):
                            gb0 = i * (2 * G)
                            gb1 = gb0 + G
                            for kk in range(G // 16):
                                gi_v[pl.ds(kk * 16, 16)] = plsc.load_gather(
                                    cr_v,
                                    [gb0 + kk * 16 + iota]) + b * tbl_rows
                            cp0 = pltpu.async_copy(tab.at[gi_v], rows_v, sg)
                            for kk in range(G // 16):
                                gi2_v[pl.ds(kk * 16, 16)] = (
                                    plsc.load_gather(
                                        cr_v, [gb1 + kk * 16 + iota])
                                    + b * tbl_rows)
                            cp1 = pltpu.async_copy(tab.at[gi2_v], rows2_v,
                                                   sg2)
                            cp0.wait()
                            accum(gb0, rows_v)
                            cp1.wait()
                            accum(gb1, rows2_v)
                            return 0

                        lax.fori_loop(0, ning >> 1, grp2, 0)
                        return 0

                    lax.fori_loop(0, nblk, blkf, 0)

                obase = pl.multiple_of(
                    (b * out_rows + (p * NW + w) * R_ROWS) * H, 8)
                pltpu.sync_copy(
                    acc_v.at[pl.ds(0, R_ROWS * H)],
                    out_hbm.at[pl.ds(obase, R_ROWS * H)])
                return 0

            lax.fori_loop(0, B, per_b, 0)

    dummy_i = jnp.zeros((8,), jnp.int32)
    dummy_f = jnp.zeros((8,), jnp.float32)
    args = list(tabs)
    for (mask_arr, rid_arr, w_arr, _, _, _) in phases:
        args += [mask_arr,
                 rid_arr if rid_arr is not None else dummy_i,
                 w_arr if w_arr is not None else dummy_f]
    return k(*args)[0]


def kernel(h, edge_index, b2_edges, b2_cells, We1, be1, We2, be2,
           Wc1, bc1, Wc2, bc2, Wn1, bn1, Wn2, bn2, Wg, bg):
    src = edge_index[0]
    dst = edge_index[1]

    deg_nodes = jnp.maximum(
        jnp.zeros((N,), jnp.float32).at[src].add(1.0).at[dst].add(1.0), 1.0)
    deg_cells = jnp.maximum(
        jnp.zeros((C,), jnp.float32).at[b2_cells].add(1.0), 1.0)
    deg_edges_c = jnp.maximum(
        jnp.zeros((E,), jnp.float32).at[b2_edges].add(1.0), 1.0)
    inv_dn = (1.0 / deg_nodes)[:, None]
    inv_dc = 1.0 / deg_cells
    wp = (1.0 / deg_edges_c)[b2_edges]
    pd = dst[b2_edges]
    ps = src[b2_edges]

    pad_iota = jnp.arange(E_PAD - E, dtype=jnp.int32)
    padN = lambda a: jnp.concatenate([a, pad_iota % N])
    padS = lambda a: jnp.concatenate(
        [a, jnp.full((E_PAD - E,), SENT, jnp.int32)])
    pad0i = lambda a: jnp.concatenate(
        [a, jnp.zeros((E_PAD - E,), jnp.int32)])
    pad0f = lambda a: jnp.concatenate(
        [a, jnp.zeros((E_PAD - E,), jnp.float32)])

    dst_p, src_p = padN(dst), padN(src)
    dst_m, src_m = padS(dst), padS(src)
    cells_m = padS(b2_cells)
    edges_r = pad0i(b2_edges)
    cells_r = pad0i(b2_cells)
    pd_m, ps_m = padS(pd), padS(ps)
    wp_p = pad0f(wp)

    b1r = lambda v: v[None, :]

    x_e = _edge_gather(h.reshape(B * N, H), dst_p, src_p)
    ones = jnp.ones((B * E_PAD, 1), jnp.float32)
    e_mlp = _fused_mlp(x_e.reshape(B * E_PAD, H), ones, We1, b1r(be1),
                       We2, b1r(be2))

    c_raw = _accumulate(
        [e_mlp], [(cells_m, edges_r, None, 1.0, 0, E_PAD)], 4, C_PAD,
        "cells")
    inv_dc_p = jnp.concatenate(
        [inv_dc, jnp.ones((C_PAD - C,), jnp.float32)])
    c_mlp = _fused_mlp(c_raw.reshape(B * C_PAD, H),
                       jnp.tile(inv_dc_p[:, None], (B, 1)),
                       Wc1, b1r(bc1), Wc2, b1r(bc2))

    hm = _accumulate(
        [e_mlp, c_mlp],
        [(dst_m, None, None, 1.0, 0, E_PAD),
         (src_m, None, None, -1.0, 0, E_PAD),
         (pd_m, cells_r, wp_p, 1.0, 1, C_PAD),
         (ps_m, cells_r, wp_p, -1.0, 1, C_PAD)],
        1, N_PAD, "nodes")
    hm_raw = hm.reshape(B, N_PAD, H)[:, :N, :]

    out = _fused_tail(hm_raw.reshape(B * N, H), jnp.tile(inv_dn, (B, 1)),
                      h.reshape(B * N, H), Wn1, b1r(bn1), Wn2, b1r(bn2),
                      Wg[:H], Wg[H:], b1r(bg))
    return out.reshape(B, N, H)
